# Initial kernel scaffold; baseline (speedup 1.0000x reference)
import jax
import jax.numpy as jnp
from jax.experimental import pallas as pl


def kernel(x, edge_index, batch, Wl1, bl1, Wr1, br1, att1, bias1, Wl2, bl2, Wr2, br2, att2, bias2, Wc, bc):
    raise NotImplementedError("write your pallas kernel here")


# speedup vs baseline: 4.1828x; 4.1828x over previous
import functools

import jax
import jax.numpy as jnp
from jax import lax
from jax.experimental import pallas as pl
from jax.experimental.pallas import tpu as pltpu
from jax.experimental.pallas import tpu_sc as plsc

N = 10000
E = 160000
FIN = 128
H = 16
C = 32
HC = H * C
G = 64
NCLS = 16

NCORES = 2
NSUB = 16
NTILES = NCORES * NSUB
EPT = E // NTILES
K = 40
NG = EPT // K
NCC = 4
CW = HC // NCC
HPC = H // NCC

RB = 1000
NRB = N // RB

_MESH = plsc.VectorSubcoreMesh(core_axis_name="c", subcore_axis_name="s")


def _lin_pair_body(x_ref, wl_ref, bl_ref, wr_ref, br_ref, xlf_ref, xlc_ref,
                   xrf_ref):
    x = x_ref[...]
    xl = jnp.dot(x, wl_ref[...], preferred_element_type=jnp.float32) + bl_ref[...]
    xr = jnp.dot(x, wr_ref[...], preferred_element_type=jnp.float32) + br_ref[...]
    xlf_ref[...] = xl
    xrf_ref[...] = xr
    xlc_ref[...] = xl.reshape(RB, NCC, CW).transpose(1, 0, 2)


def _lin_pair(x, Wl, bl, Wr, br):
    kin = x.shape[1]
    return pl.pallas_call(
        _lin_pair_body,
        grid=(NRB,),
        in_specs=[
            pl.BlockSpec((RB, kin), lambda i: (i, 0)),
            pl.BlockSpec((kin, HC), lambda i: (0, 0)),
            pl.BlockSpec((1, HC), lambda i: (0, 0)),
            pl.BlockSpec((kin, HC), lambda i: (0, 0)),
            pl.BlockSpec((1, HC), lambda i: (0, 0)),
        ],
        out_specs=[
            pl.BlockSpec((RB, HC), lambda i: (i, 0)),
            pl.BlockSpec((NCC, RB, CW), lambda i: (0, i, 0)),
            pl.BlockSpec((RB, HC), lambda i: (i, 0)),
        ],
        out_shape=[
            jax.ShapeDtypeStruct((N, HC), jnp.float32),
            jax.ShapeDtypeStruct((NCC, N, CW), jnp.float32),
            jax.ShapeDtypeStruct((N, HC), jnp.float32),
        ],
    )(x, Wl, bl.reshape(1, HC), Wr, br.reshape(1, HC))


def _combine_block(part_ref, den_ref, bias_ref):
    p = part_ref[0] + part_ref[1]
    p = p.transpose(1, 0, 2).reshape(RB, HC)
    den = jnp.maximum(den_ref[0] + den_ref[1], 1e-16)
    dene = jnp.broadcast_to(den[:, :, None], (RB, H, C)).reshape(RB, HC)
    out = p / dene
    h = out.reshape(RB, H, C).mean(axis=1) + bias_ref[...]
    return jnp.where(h > 0, h, 0.01 * h)


def _combine_lin_body(part_ref, den_ref, bias_ref, wl_ref, bl_ref, wr_ref,
                      br_ref, xlf_ref, xlc_ref, xrf_ref):
    h = _combine_block(part_ref, den_ref, bias_ref)
    xl = jnp.dot(h, wl_ref[...], preferred_element_type=jnp.float32) + bl_ref[...]
    xr = jnp.dot(h, wr_ref[...], preferred_element_type=jnp.float32) + br_ref[...]
    xlf_ref[...] = xl
    xrf_ref[...] = xr
    xlc_ref[...] = xl.reshape(RB, NCC, CW).transpose(1, 0, 2)


def _combine_lin(part, den, bias, Wl, bl, Wr, br):
    return pl.pallas_call(
        _combine_lin_body,
        grid=(NRB,),
        in_specs=[
            pl.BlockSpec((2, NCC, RB, CW), lambda i: (0, 0, i, 0)),
            pl.BlockSpec((2, RB, H), lambda i: (0, i, 0)),
            pl.BlockSpec((1, C), lambda i: (0, 0)),
            pl.BlockSpec((C, HC), lambda i: (0, 0)),
            pl.BlockSpec((1, HC), lambda i: (0, 0)),
            pl.BlockSpec((C, HC), lambda i: (0, 0)),
            pl.BlockSpec((1, HC), lambda i: (0, 0)),
        ],
        out_specs=[
            pl.BlockSpec((RB, HC), lambda i: (i, 0)),
            pl.BlockSpec((NCC, RB, CW), lambda i: (0, i, 0)),
            pl.BlockSpec((RB, HC), lambda i: (i, 0)),
        ],
        out_shape=[
            jax.ShapeDtypeStruct((N, HC), jnp.float32),
            jax.ShapeDtypeStruct((NCC, N, CW), jnp.float32),
            jax.ShapeDtypeStruct((N, HC), jnp.float32),
        ],
    )(part, den, bias.reshape(1, C), Wl, bl.reshape(1, HC), Wr, br.reshape(1, HC))


def _final_body(part_ref, den_ref, bias_ref, batch_ref, wc_ref, bc_ref,
                out_ref, acc_s, acc_c):
    i = pl.program_id(0)

    @pl.when(i == 0)
    def _():
        acc_s[...] = jnp.zeros_like(acc_s)
        acc_c[...] = jnp.zeros_like(acc_c)

    h = _combine_block(part_ref, den_ref, bias_ref)
    rows = lax.broadcasted_iota(jnp.int32, (G, RB), 0)
    bb = jnp.broadcast_to(batch_ref[0], (G, RB))
    oht = (bb == rows).astype(jnp.float32)
    acc_s[...] += jnp.dot(oht, h, preferred_element_type=jnp.float32)
    acc_c[...] += jnp.dot(oht, jnp.ones((RB, C), jnp.float32),
                          preferred_element_type=jnp.float32)

    @pl.when(i == NRB - 1)
    def _():
        pooled = acc_s[...] / jnp.maximum(acc_c[...], 1.0)
        out_ref[...] = jnp.dot(pooled, wc_ref[...],
                               preferred_element_type=jnp.float32) + bc_ref[...]


def _final(part, den, bias, batch3, Wc, bc):
    return pl.pallas_call(
        _final_body,
        grid=(NRB,),
        in_specs=[
            pl.BlockSpec((2, NCC, RB, CW), lambda i: (0, 0, i, 0)),
            pl.BlockSpec((2, RB, H), lambda i: (0, i, 0)),
            pl.BlockSpec((1, C), lambda i: (0, 0)),
            pl.BlockSpec((1, 1, RB), lambda i: (i, 0, 0)),
            pl.BlockSpec((C, NCLS), lambda i: (0, 0)),
            pl.BlockSpec((1, NCLS), lambda i: (0, 0)),
        ],
        out_specs=pl.BlockSpec((G, NCLS), lambda i: (0, 0)),
        out_shape=jax.ShapeDtypeStruct((G, NCLS), jnp.float32),
        scratch_shapes=[
            pltpu.VMEM((G, C), jnp.float32),
            pltpu.VMEM((G, C), jnp.float32),
        ],
    )(part, den, bias.reshape(1, C), batch3, Wc, bc.reshape(1, NCLS))


def _p1_body(xl_hbm, xr_hbm, src_hbm, dst_hbm, attT_hbm, z16_hbm,
             ex_out, den_out,
             idx_s, idx_d, rows_l, rows_r, ex_buf, att_v, den_sh, sem1, sem2):
    cid = lax.axis_index("c")
    sid = lax.axis_index("s")
    wid = cid * NSUB + sid

    pltpu.sync_copy(attT_hbm, att_v)

    @pl.when(sid == 0)
    def _():
        pltpu.sync_copy(z16_hbm, den_sh)

    plsc.subcore_barrier()

    lanes = lax.iota(jnp.int32, 16)
    colbase = lanes * C

    def chunk_body(g, carry):
        base = wid * EPT + g * K
        pltpu.sync_copy(src_hbm.at[pl.ds(base, K)], idx_s)
        pltpu.sync_copy(dst_hbm.at[pl.ds(base, K)], idx_d)
        cpl = pltpu.async_copy(xl_hbm.at[idx_s], rows_l, sem1)
        cpr = pltpu.async_copy(xr_hbm.at[idx_d], rows_r, sem2)
        cpl.wait()
        cpr.wait()

        def edge_body(k, carry2):
            ksplat = jnp.full((16,), 0, jnp.int32) + k
            acc = jnp.zeros((16,), jnp.float32)
            for c in range(C):
                colc = colbase + c
                vl = plsc.load_gather(rows_l, [ksplat, colc])
                vr = plsc.load_gather(rows_r, [ksplat, colc])
                s = vl + vr
                s = jnp.where(s > 0, s, 0.2 * s)
                acc = acc + s * att_v[c, :]
            ex = jnp.exp(acc)
            plsc.store_scatter(ex_buf, [ksplat, lanes], ex)
            return carry2

        lax.fori_loop(0, K, edge_body, 0)
        pltpu.sync_copy(ex_buf, ex_out.at[pl.ds(base, K)])
        pltpu.sync_copy(ex_buf, den_sh.at[idx_d], add=True)
        return carry

    lax.fori_loop(0, NG, chunk_body, 0)
    plsc.subcore_barrier()

    @pl.when(sid == 0)
    def _():
        pltpu.sync_copy(den_sh, den_out.at[cid])


def _sc_scores(xlf, xrf, src, dst, attT, zeros16):
    fn = functools.partial(
        pl.kernel,
        out_type=(
            jax.ShapeDtypeStruct((E, H), jnp.float32),
            jax.ShapeDtypeStruct((NCORES, N, H), jnp.float32),
        ),
        mesh=_MESH,
        scratch_types=[
            pltpu.VMEM((K,), jnp.int32),
            pltpu.VMEM((K,), jnp.int32),
            pltpu.VMEM((K, HC), jnp.float32),
            pltpu.VMEM((K, HC), jnp.float32),
            pltpu.VMEM((K, H), jnp.float32),
            pltpu.VMEM((C, H), jnp.float32),
            pltpu.VMEM_SHARED((N, H), jnp.float32),
            pltpu.SemaphoreType.DMA,
            pltpu.SemaphoreType.DMA,
        ],
        compiler_params=pltpu.CompilerParams(needs_layout_passes=False),
    )(_p1_body)
    return fn(xlf, xrf, src, dst, attT, zeros16)


def _p3_body(xc0, xc1, xc2, xc3, ex_hbm, src_hbm, dst_hbm, z128_hbm,
             out_hbm,
             idx_s, idx_d, rows, wrows, ex_v, out_sh, sem1):
    cid = lax.axis_index("c")
    sid = lax.axis_index("s")
    wid = cid * NSUB + sid
    lanes = lax.iota(jnp.int32, 16)

    for cc, xc in enumerate((xc0, xc1, xc2, xc3)):
        @pl.when(sid == 0)
        def _():
            pltpu.sync_copy(z128_hbm, out_sh)

        plsc.subcore_barrier()

        def chunk_body(g, carry, xc=xc, cc=cc):
            base = wid * EPT + g * K
            pltpu.sync_copy(src_hbm.at[pl.ds(base, K)], idx_s)
            pltpu.sync_copy(dst_hbm.at[pl.ds(base, K)], idx_d)
            pltpu.sync_copy(ex_hbm.at[pl.ds(base, K)], ex_v)
            pltpu.async_copy(xc.at[idx_s], rows, sem1).wait()

            def edge_body(k, carry2):
                ksplat = jnp.full((16,), 0, jnp.int32) + k
                for hh in range(HPC):
                    hsplat = jnp.full((16,), cc * HPC + hh, jnp.int32)
                    w = plsc.load_gather(ex_v, [ksplat, hsplat])
                    for j in range(2):
                        colc = lanes + (hh * C + j * 16)
                        v = plsc.load_gather(rows, [ksplat, colc])
                        plsc.store_scatter(wrows, [ksplat, colc], v * w)
                return carry2

            lax.fori_loop(0, K, edge_body, 0)
            pltpu.sync_copy(wrows, out_sh.at[idx_d], add=True)
            return carry

        lax.fori_loop(0, NG, chunk_body, 0)
        plsc.subcore_barrier()

        @pl.when(sid == 0)
        def _():
            pltpu.sync_copy(out_sh, out_hbm.at[cid, cc])

        plsc.subcore_barrier()


def _sc_aggregate(xlc, ex, src, dst, zeros128):
    fn = functools.partial(
        pl.kernel,
        out_type=jax.ShapeDtypeStruct((NCORES, NCC, N, CW), jnp.float32),
        mesh=_MESH,
        scratch_types=[
            pltpu.VMEM((K,), jnp.int32),
            pltpu.VMEM((K,), jnp.int32),
            pltpu.VMEM((K, CW), jnp.float32),
            pltpu.VMEM((K, CW), jnp.float32),
            pltpu.VMEM((K, H), jnp.float32),
            pltpu.VMEM_SHARED((N, CW), jnp.float32),
            pltpu.SemaphoreType.DMA,
        ],
        compiler_params=pltpu.CompilerParams(needs_layout_passes=False),
    )(_p3_body)
    return fn(xlc[0], xlc[1], xlc[2], xlc[3], ex, src, dst, zeros128)


def kernel(x, edge_index, batch, Wl1, bl1, Wr1, br1, att1, bias1,
           Wl2, bl2, Wr2, br2, att2, bias2, Wc, bc):
    src = edge_index[0]
    dst = edge_index[1]
    zeros16 = jnp.zeros((N, H), jnp.float32)
    zeros128 = jnp.zeros((N, CW), jnp.float32)
    attT1 = att1.T
    attT2 = att2.T
    batch3 = batch.reshape(NRB, 1, RB)

    xlf, xlc, xrf = _lin_pair(x, Wl1, bl1, Wr1, br1)
    ex1, den1 = _sc_scores(xlf, xrf, src, dst, attT1, zeros16)
    part1 = _sc_aggregate([xlc[i] for i in range(NCC)], ex1, src, dst, zeros128)

    xlf2, xlc2, xrf2 = _combine_lin(part1, den1, bias1, Wl2, bl2, Wr2, br2)
    ex2, den2 = _sc_scores(xlf2, xrf2, src, dst, attT2, zeros16)
    part2 = _sc_aggregate([xlc2[i] for i in range(NCC)], ex2, src, dst, zeros128)

    return _final(part2, den2, bias2, batch3, Wc, bc)
